# baseline (speedup 1.0000x reference)
import functools

import jax
import jax.numpy as jnp
from jax.experimental import pallas as pl
from jax.experimental.pallas import tpu as pltpu


def _roi_body(fm_ref, cd_ref, sc_ref, out_ref, *, H, W):
    N = sc_ref.shape[2]
    half = fm_ref.shape[2]
    cd = cd_ref[0]
    x0 = cd[0:1, :]
    x1 = cd[1:2, :]
    y0 = cd[2:3, :]
    y1 = cd[3:4, :]

    xi = jax.lax.broadcasted_iota(jnp.int32, (W, N), 0)
    colf = jnp.where((xi >= x0) & (xi < x1), 1.0, 0.0).astype(jnp.float32)
    yi = jax.lax.broadcasted_iota(jnp.int32, (H, N), 0)
    rowf = jnp.where((yi >= y0) & (yi < y1), 1.0, 0.0).astype(jnp.float32)

    m3 = rowf[:, None, :] * colf[None, :, :]
    ind = m3.reshape(H * W, N)
    ind3 = ind.reshape(half, 2, N)
    ind_e = ind3[:, 0, :].astype(jnp.bfloat16)
    ind_o = ind3[:, 1, :].astype(jnp.bfloat16)

    wu = pltpu.bitcast(fm_ref[0], jnp.uint32)
    xlo = pltpu.bitcast(wu << 16, jnp.float32).astype(jnp.bfloat16)
    xhi = (pltpu.bitcast(wu & jnp.uint32(0xFFFF0000), jnp.float32)
           .astype(jnp.bfloat16))

    acc = (jnp.dot(xlo, ind_e, preferred_element_type=jnp.float32)
           + jnp.dot(xhi, ind_o, preferred_element_type=jnp.float32))
    out_ref[0] = acc * sc_ref[0]


def kernel(feature_map, keypoints, mask, original_H, original_W):
    B, C, H, W = feature_map.shape
    N = keypoints.shape[1]
    sx = W / original_W
    sy = H / original_H
    x, y, w, h = (keypoints[..., 0], keypoints[..., 1],
                  keypoints[..., 2], keypoints[..., 3])
    xr = jnp.clip((x * sx).astype(jnp.int32), 0, W - 1)
    yr = jnp.clip((y * sy).astype(jnp.int32), 0, H - 1)
    wr = jnp.minimum(jnp.maximum((w * sx).astype(jnp.int32), 1), W - xr)
    hr = jnp.minimum(jnp.maximum((h * sy).astype(jnp.int32), 1), H - yr)
    coords = jnp.stack([xr, xr + wr, yr, yr + hr], axis=1)
    area = (hr * wr).astype(jnp.float32)
    scale = jnp.where(mask > 0, 1.0 / area, 0.0).reshape(B, 1, N)

    half = (H * W) // 2
    fm16 = feature_map.reshape(B, C, H * W).astype(jnp.bfloat16)
    words = jax.lax.bitcast_convert_type(
        fm16.reshape(B, C, half, 2), jnp.int32)

    out = pl.pallas_call(
        functools.partial(_roi_body, H=H, W=W),
        grid=(B,),
        in_specs=[
            pl.BlockSpec((1, C, half), lambda b: (b, 0, 0)),
            pl.BlockSpec((1, 4, N), lambda b: (b, 0, 0)),
            pl.BlockSpec((1, 1, N), lambda b: (b, 0, 0)),
        ],
        out_specs=pl.BlockSpec((1, C, N), lambda b: (b, 0, 0)),
        out_shape=jax.ShapeDtypeStruct((B, C, N), jnp.float32),
        compiler_params=pltpu.CompilerParams(
            dimension_semantics=("parallel",),
            vmem_limit_bytes=50 * 1024 * 1024,
        ),
    )(words, coords, scale)
    return jnp.transpose(out, (0, 2, 1))


# speedup vs baseline: 4.5126x; 4.5126x over previous
import functools

import jax
import jax.numpy as jnp
from jax.experimental import pallas as pl
from jax.experimental.pallas import tpu as pltpu


def _roi_body(fm_ref, cd_ref, sc_ref, out_ref, *, H, W):
    N = sc_ref.shape[2]
    cd = cd_ref[0]
    x0 = cd[0:1, :]
    x1 = cd[1:2, :]
    y0 = cd[2:3, :]
    y1 = cd[3:4, :]

    xi = jax.lax.broadcasted_iota(jnp.int32, (W, N), 0)
    colf = jnp.where((xi >= x0) & (xi < x1), 1.0, 0.0).astype(jnp.float32)
    yi = jax.lax.broadcasted_iota(jnp.int32, (H, N), 0)
    rowf = jnp.where((yi >= y0) & (yi < y1), 1.0, 0.0).astype(jnp.float32)

    m3 = rowf[:, None, :] * colf[None, :, :]
    ind = m3.reshape(H * W, N).astype(jnp.bfloat16)

    acc = jnp.dot(fm_ref[0], ind, preferred_element_type=jnp.float32)
    out_ref[0] = acc * sc_ref[0]


def kernel(feature_map, keypoints, mask, original_H, original_W):
    B, C, H, W = feature_map.shape
    N = keypoints.shape[1]
    sx = W / original_W
    sy = H / original_H
    x, y, w, h = (keypoints[..., 0], keypoints[..., 1],
                  keypoints[..., 2], keypoints[..., 3])
    xr = jnp.clip((x * sx).astype(jnp.int32), 0, W - 1)
    yr = jnp.clip((y * sy).astype(jnp.int32), 0, H - 1)
    wr = jnp.minimum(jnp.maximum((w * sx).astype(jnp.int32), 1), W - xr)
    hr = jnp.minimum(jnp.maximum((h * sy).astype(jnp.int32), 1), H - yr)
    coords = jnp.stack([xr, xr + wr, yr, yr + hr], axis=1)
    area = (hr * wr).astype(jnp.float32)
    scale = jnp.where(mask > 0, 1.0 / area, 0.0).reshape(B, 1, N)

    fm = feature_map.reshape(B, C, H * W).astype(jnp.bfloat16)
    out = pl.pallas_call(
        functools.partial(_roi_body, H=H, W=W),
        grid=(B,),
        in_specs=[
            pl.BlockSpec((1, C, H * W), lambda b: (b, 0, 0)),
            pl.BlockSpec((1, 4, N), lambda b: (b, 0, 0)),
            pl.BlockSpec((1, 1, N), lambda b: (b, 0, 0)),
        ],
        out_specs=pl.BlockSpec((1, C, N), lambda b: (b, 0, 0)),
        out_shape=jax.ShapeDtypeStruct((B, C, N), jnp.float32),
        compiler_params=pltpu.CompilerParams(
            dimension_semantics=("parallel",),
            vmem_limit_bytes=50 * 1024 * 1024,
        ),
    )(fm, coords, scale)
    return jnp.transpose(out, (0, 2, 1))
